# baseline (speedup 1.0000x reference)
import functools

import jax
import jax.numpy as jnp
import numpy as np
from jax import lax
from jax.experimental import pallas as pl
from jax.experimental.pallas import tpu as pltpu
from jax.experimental.pallas import tpu_sc as plsc

HID = 256
HEADS = 8
DH = HID // HEADS
G = 16

_NEG = -3.0e38

_M16 = np.zeros((HID, 16), np.float32)
for _h in range(HEADS):
    _M16[_h * DH:(_h + 1) * DH, _h] = 1.0
_M16S = _M16 / np.sqrt(DH)
_E16 = _M16.T.copy()


_SC_CHUNK = 128


@functools.lru_cache(maxsize=None)
def _sc_gather_fn(V, D, B):
    info = plsc.get_sparse_core_info()
    NC, NS = info.num_cores, info.num_subcores
    NW = NC * NS
    b_per_w = B // NW
    n_chunks = b_per_w // _SC_CHUNK
    mesh = plsc.VectorSubcoreMesh(core_axis_name="c", subcore_axis_name="s")

    @functools.partial(
        pl.kernel, mesh=mesh,
        out_type=jax.ShapeDtypeStruct((B, D), jnp.float32),
        scratch_types=[
            pltpu.VMEM((_SC_CHUNK,), jnp.int32),
            pltpu.VMEM((_SC_CHUNK, D), jnp.float32),
            pltpu.SemaphoreType.DMA,
        ],
    )
    def k(table_hbm, idx_hbm, out_hbm, idx_v, rows_v, sem):
        wid = lax.axis_index("s") * NC + lax.axis_index("c")
        base = wid * b_per_w

        def body(i, _):
            off = base + i * _SC_CHUNK
            pltpu.sync_copy(idx_hbm.at[pl.ds(off, _SC_CHUNK)], idx_v)
            pltpu.async_copy(table_hbm.at[idx_v], rows_v, sem).wait()
            pltpu.sync_copy(rows_v, out_hbm.at[pl.ds(off, _SC_CHUNK)])
            return 0

        lax.fori_loop(0, n_chunks, body, 0)

    return k


def _sc_gather(table, idx):
    V, D = table.shape
    B0 = idx.shape[0]
    B = ((B0 + 4095) // 4096) * 4096
    if B != B0:
        idx = jnp.concatenate([idx, jnp.zeros((B - B0,), jnp.int32)])
    out = _sc_gather_fn(V, D, B)(table, idx)
    return out[:B0]


_TM = 512


def _padrows(x, tm, val=0.0):
    m = x.shape[0]
    mp = ((m + tm - 1) // tm) * tm
    if mp == m:
        return x
    return jnp.pad(x, ((0, mp - m),) + ((0, 0),) * (x.ndim - 1),
                   constant_values=val)


def _ew(fn, *arrays):
    M = max(a.shape[0] for a in arrays)
    tm = min(1024, ((M + 7) // 8) * 8)
    padded = []
    specs = []
    shapes = []
    nblk = 1
    for a in arrays:
        if a.shape[0] == M:
            ap = _padrows(a, tm)
            nblk = ap.shape[0] // tm
            padded.append(ap)
            specs.append(pl.BlockSpec((tm, a.shape[1]), lambda i: (i, 0)))
            shapes.append((tm, a.shape[1]))
        else:
            padded.append(a)
            specs.append(pl.BlockSpec(a.shape, lambda i: (0, 0)))
            shapes.append(a.shape)
    oshape = jax.eval_shape(fn, *[
        jax.ShapeDtypeStruct(s, jnp.float32) for s in shapes])

    def body(*refs):
        ins = refs[:-1]
        out = refs[-1]
        out[...] = fn(*[r[...] for r in ins])

    res = pl.pallas_call(
        body,
        grid=(nblk,),
        in_specs=specs,
        out_specs=pl.BlockSpec((tm, oshape.shape[1]), lambda i: (i, 0)),
        out_shape=jax.ShapeDtypeStruct((nblk * tm, oshape.shape[1]),
                                       jnp.float32),
    )(*padded)
    return res[:M]


def _mm(x, wt, b=None, act=None):
    M, K = x.shape
    N = wt.shape[1]
    tm = min(_TM, ((M + 7) // 8) * 8)
    xp = _padrows(x, tm)
    nblk = xp.shape[0] // tm
    if b is None:
        b = jnp.zeros((1, N), jnp.float32)
    else:
        b = b.reshape(1, N)

    def body(x_ref, w_ref, b_ref, o_ref):
        y = jnp.dot(x_ref[...], w_ref[...],
                    preferred_element_type=jnp.float32) + b_ref[...]
        if act is not None:
            y = act(y)
        o_ref[...] = y

    res = pl.pallas_call(
        body,
        grid=(nblk,),
        in_specs=[pl.BlockSpec((tm, K), lambda i: (i, 0)),
                  pl.BlockSpec((K, N), lambda i: (0, 0)),
                  pl.BlockSpec((1, N), lambda i: (0, 0))],
        out_specs=pl.BlockSpec((tm, N), lambda i: (i, 0)),
        out_shape=jax.ShapeDtypeStruct((nblk * tm, N), jnp.float32),
    )(xp, wt, b)
    return res[:M]


def _segscan(v, ids, kind, rev=False):
    M, D = v.shape
    tm = 4096 if D <= 16 else 1024
    fill = 0.0 if kind == "add" else _NEG
    op = (lambda a, b: a + b) if kind == "add" else jnp.maximum
    vp = _padrows(v, tm, val=fill)
    idp = _padrows(ids, tm, val=np.int32(2147483647))
    nblk = vp.shape[0] // tm

    def imap(i):
        return ((nblk - 1 - i) if rev else i, 0)

    def body(v_ref, id_ref, o_ref, cm_ref, ci_ref):
        pid = pl.program_id(0)

        @pl.when(pid == 0)
        def _():
            cm_ref[...] = jnp.full((1, D), fill, jnp.float32)
            ci_ref[...] = jnp.full((1, 1), -1, jnp.int32)

        x = v_ref[...]
        sid = id_ref[...]
        k = 1
        while k < tm:
            if rev:
                xs = jnp.concatenate(
                    [x[k:], jnp.full((k, D), fill, jnp.float32)], axis=0)
                ss = jnp.concatenate(
                    [sid[k:], jnp.full((k, 1), -1, jnp.int32)], axis=0)
            else:
                xs = jnp.concatenate(
                    [jnp.full((k, D), fill, jnp.float32), x[:-k]], axis=0)
                ss = jnp.concatenate(
                    [jnp.full((k, 1), -1, jnp.int32), sid[:-k]], axis=0)
            x = jnp.where(sid == ss, op(x, xs), x)
            k *= 2
        x = jnp.where(sid == ci_ref[...], op(x, cm_ref[...]), x)
        o_ref[...] = x
        if rev:
            cm_ref[...] = x[0:1, :]
            ci_ref[...] = sid[0:1, :]
        else:
            cm_ref[...] = x[tm - 1:tm, :]
            ci_ref[...] = sid[tm - 1:tm, :]

    res = pl.pallas_call(
        body,
        grid=(nblk,),
        in_specs=[pl.BlockSpec((tm, D), imap),
                  pl.BlockSpec((tm, 1), imap)],
        out_specs=pl.BlockSpec((tm, D), imap),
        out_shape=jax.ShapeDtypeStruct((nblk * tm, D), jnp.float32),
        scratch_shapes=[pltpu.VMEM((1, D), jnp.float32),
                        pltpu.VMEM((1, 1), jnp.int32)],
    )(vp, idp)
    return res[:M]


def _colreduce(x, mode):
    M, D = x.shape
    tm = _TM
    vp = _padrows(x, tm, val=(0.0 if mode == "sum2" else _NEG))
    nblk = vp.shape[0] // tm

    def body(v_ref, o_ref):
        pid = pl.program_id(0)

        @pl.when(pid == 0)
        def _():
            o_ref[...] = jnp.full(
                (8, D), 0.0 if mode == "sum2" else _NEG, jnp.float32)

        blk = v_ref[...]
        if mode == "sum2":
            o_ref[0:1, :] = o_ref[0:1, :] + jnp.sum(blk, 0, keepdims=True)
            o_ref[1:2, :] = o_ref[1:2, :] + jnp.sum(blk * blk, 0,
                                                    keepdims=True)
        else:
            o_ref[0:1, :] = jnp.maximum(o_ref[0:1, :],
                                        jnp.max(blk, 0, keepdims=True))

    res = pl.pallas_call(
        body,
        grid=(nblk,),
        in_specs=[pl.BlockSpec((tm, D), lambda i: (i, 0))],
        out_specs=pl.BlockSpec((8, D), lambda i: (0, 0)),
        out_shape=jax.ShapeDtypeStruct((8, D), jnp.float32),
    )(vp)
    return res


def _gate_kernel(out, xr, wbo, wbx):
    M = out.shape[0]
    tm = _TM
    op = _padrows(out, tm)
    xp = _padrows(xr, tm)
    nblk = op.shape[0] // tm

    def body(o_ref, x_ref, wo_ref, wx_ref, y_ref):
        o = o_ref[...]
        x = x_ref[...]
        z = (jnp.dot(o, wo_ref[...], preferred_element_type=jnp.float32)
             + jnp.dot(x, wx_ref[...], preferred_element_type=jnp.float32))
        beta = jax.nn.sigmoid(z[:, 0:1])
        y_ref[...] = beta * x + (1.0 - beta) * o

    res = pl.pallas_call(
        body,
        grid=(nblk,),
        in_specs=[pl.BlockSpec((tm, HID), lambda i: (i, 0)),
                  pl.BlockSpec((tm, HID), lambda i: (i, 0)),
                  pl.BlockSpec((HID, 128), lambda i: (0, 0)),
                  pl.BlockSpec((HID, 128), lambda i: (0, 0))],
        out_specs=pl.BlockSpec((tm, HID), lambda i: (i, 0)),
        out_shape=jax.ShapeDtypeStruct((nblk * tm, HID), jnp.float32),
    )(op, xp, wbo, wbx)
    return res[:M]


def _seg_sum(vals, ids2d, end_idx, mask_end):
    sc = _segscan(vals, ids2d, "add")
    rows = _sc_gather(sc, end_idx)
    return _ew(lambda a, m: a * m, rows, mask_end)


def _softmax_norm(alpha16, dst2d):
    mf = _segscan(alpha16, dst2d, "max", rev=False)
    mb = _segscan(alpha16, dst2d, "max", rev=True)
    ex = _ew(lambda a, f, b: jnp.exp(a - jnp.maximum(f, b)),
             alpha16, mf, mb)
    sf = _segscan(ex, dst2d, "add", rev=False)
    sb = _segscan(ex, dst2d, "add", rev=True)
    return _ew(lambda e, f, b: e / (f + b - e + 1e-16), ex, sf, sb)


def kernel(x, edge_index, edge_attr, batch, params):
    p = params
    N = x.shape[0]
    E = edge_index.shape[1]
    mask16 = jnp.asarray(_M16)
    mask16s = jnp.asarray(_M16S)
    expand = jnp.asarray(_E16)

    dst0 = edge_index[1]
    perm = jnp.argsort(dst0).astype(jnp.int32)
    dst = dst0[perm]
    srcp = edge_index[0][perm]
    end = jnp.searchsorted(dst, jnp.arange(N, dtype=jnp.int32),
                           side="right").astype(jnp.int32)
    start = jnp.concatenate([jnp.zeros((1,), jnp.int32), end[:-1]])
    idx_pair = {
        "end_idx": jnp.maximum(end - 1, 0),
        "mask_end": (end > start).astype(jnp.float32).reshape(N, 1),
    }
    dst2d = dst.reshape(E, 1)

    bend = jnp.searchsorted(batch, jnp.arange(G, dtype=jnp.int32),
                            side="right").astype(jnp.int32)
    bstart = jnp.concatenate([jnp.zeros((1,), jnp.int32), bend[:-1]])
    bpair = {
        "end_idx": jnp.maximum(bend - 1, 0),
        "mask_end": (bend > bstart).astype(jnp.float32).reshape(G, 1),
    }
    batch2d = batch.astype(jnp.int32).reshape(N, 1)
    cnt = (bend - bstart).astype(jnp.float32).reshape(G, 1)

    xin = jnp.pad(x, ((0, 0), (0, 256 - x.shape[1])))
    encWt = jnp.pad(p["encW"].T, ((0, 256 - x.shape[1]), (0, 0)))
    h = _mm(xin, encWt, p["encb"])

    ea16 = jnp.pad(edge_attr, ((0, 0), (0, 16 - edge_attr.shape[1])))
    eencWt = jnp.pad(p["eencW"].T, ((0, 16 - edge_attr.shape[1]), (0, 0)))
    ea0 = _mm(ea16, eencWt, p["eencb"])
    ea = _sc_gather(ea0, perm)

    def tconv(xc, pre):
        q = _mm(xc, p[pre + "Wq"].T, p[pre + "bq"])
        k = _mm(xc, p[pre + "Wk"].T, p[pre + "bk"])
        v = _mm(xc, p[pre + "Wv"].T, p[pre + "bv"])
        e = _mm(ea, p[pre + "We"].T)
        kv = jnp.concatenate([k, v], axis=1)
        kvs = _sc_gather(kv, srcp)
        ks, vs = kvs[:, :HID], kvs[:, HID:]
        qd = _sc_gather(q, dst)
        alpha16 = _ew(lambda a, b, c, mk: jnp.dot(
            (a * (b + c)), mk, preferred_element_type=jnp.float32),
            qd, ks, e, mask16s)
        alphan = _softmax_norm(alpha16, dst2d)
        wv = _ew(lambda vv, ee, an, ex16: (vv + ee) * jnp.dot(
            an, ex16, preferred_element_type=jnp.float32),
            vs, e, alphan, expand)
        out = _seg_sum(wv, dst2d, idx_pair["end_idx"],
                       idx_pair["mask_end"])
        xr = _mm(xc, p[pre + "Ws"].T, p[pre + "bs"])
        wb = p[pre + "Wb"]
        wbo = jnp.pad((wb[:, :HID] + wb[:, 2 * HID:]).T, ((0, 0), (0, 127)))
        wbx = jnp.pad((wb[:, HID:2 * HID] - wb[:, 2 * HID:]).T,
                      ((0, 0), (0, 127)))
        return _gate_kernel(out, xr, wbo, wbx)

    def graphnorm(xc, w, b, ms):
        ss = _colreduce(xc, "sum2")
        m = ss[0] / xc.shape[0]
        ex2 = ss[1] / xc.shape[0]
        var = ex2 - 2.0 * ms * m * m + ms * ms * m * m
        scale = w / jnp.sqrt(var + 1e-5)
        shift = b - ms * m * scale
        return _ew(lambda u, a, c: u * a + c,
                   xc, scale.reshape(1, HID), shift.reshape(1, HID))

    def gt_layer(xc, pre):
        xa = tconv(xc, pre)
        x1 = _ew(lambda a, b: a + b, xc, xa)
        xg = graphnorm(x1, p[pre + "nw"], p[pre + "nb"], p[pre + "nms"])
        f = _mm(xg, p[pre + "F1"].T, p[pre + "f1"], act=jax.nn.gelu)
        f2 = _mm(f, p[pre + "F2"].T, p[pre + "f2"])
        return _ew(lambda a, b: a + b, xg, f2)

    def gat(xc, pre):
        hx = _mm(xc, p[pre + "W"].T)
        e = _mm(ea, p[pre + "We"].T)
        asf = p[pre + "as"].reshape(1, HID)
        adf = p[pre + "ad"].reshape(1, HID)
        aef = p[pre + "ae"].reshape(1, HID)
        hdot = lambda hh, aa, mk: jnp.dot(
            hh * aa, mk, preferred_element_type=jnp.float32)
        asn = _ew(hdot, hx, asf, mask16)
        adn = _ew(hdot, hx, adf, mask16)
        aen = _ew(hdot, e, aef, mask16)
        src_tab = jnp.concatenate(
            [hx, jnp.pad(asn, ((0, 0), (0, 112)))], axis=1)
        g = _sc_gather(src_tab, srcp)
        hxs = g[:, :HID]
        g_s = g[:, HID:HID + 16]
        g_d = _sc_gather(jnp.pad(adn, ((0, 0), (0, 112))), dst)[:, :16]
        alpha16 = _ew(lambda a, b, c: jax.nn.leaky_relu(a + b + c, 0.2),
                      g_s, g_d, aen)
        alphan = _softmax_norm(alpha16, dst2d)
        wv = _ew(lambda vv, an, ex16: vv * jnp.dot(
            an, ex16, preferred_element_type=jnp.float32),
            hxs, alphan, expand)
        out = _seg_sum(wv, dst2d, idx_pair["end_idx"],
                       idx_pair["mask_end"])
        return _ew(lambda a, b: a + b, out, p[pre + "bias"].reshape(1, HID))

    def gin(xc, pre):
        xs = _sc_gather(xc, srcp)
        agg = _seg_sum(xs, dst2d, idx_pair["end_idx"],
                       idx_pair["mask_end"])
        eps11 = (1.0 + p[pre + "eps"]).reshape(1, 1)
        z = _ew(lambda a, b, c: a * c + b, xc, agg, eps11)
        z = _mm(z, p[pre + "W1"].T, p[pre + "b1"])
        ss = _colreduce(z, "sum2")
        m = ss[0] / z.shape[0]
        var = ss[1] / z.shape[0] - m * m
        scale = p[pre + "bnw"] / jnp.sqrt(var + 1e-5)
        shift = p[pre + "bnb"] - m * scale
        z = _ew(lambda u, a, c: jnp.maximum(u * a + c, 0.0),
                z, scale.reshape(1, 2 * HID), shift.reshape(1, 2 * HID))
        return _mm(z, p[pre + "W2"].T, p[pre + "b2"])

    for i in range(6):
        pre = "L%d_" % i
        if i % 3 == 0:
            xn = gt_layer(h, pre)
        elif i % 3 == 1:
            xn = gat(h, pre)
        else:
            xn = gin(h, pre)
        h = _ew(lambda a, b: a + b, h, xn)
        h = graphnorm(h, p["N%d_w" % i], p["N%d_b" % i], p["N%d_ms" % i])

    psum = _seg_sum(h, batch2d, bpair["end_idx"], bpair["mask_end"])
    pmean = _ew(lambda a, c: a / jnp.maximum(c, 1.0), psum, cnt)
    mscan = _segscan(h, batch2d, "max", rev=False)
    pmax = _sc_gather(mscan, bpair["end_idx"])
    pmax = _ew(lambda m, msk: jnp.where(msk > 0, m, 0.0),
               pmax, bpair["mask_end"])

    t = _mm(h, p["A1"].T, p["a1"], act=jnp.tanh)
    a2t = jnp.pad(p["A2"].T, ((0, 0), (0, 127)))
    a2b = jnp.pad(p["a2"].reshape(1, 1), ((0, 0), (0, 127)))
    spre = _mm(t, a2t, a2b[0])
    cm = _colreduce(spre, "max")[0:1]
    exs = _ew(lambda a, m: jnp.exp(a - m), spre, cm)
    ssum = _colreduce(exs, "sum2")[0:1]
    sw = _ew(lambda a, s: a / s, exs, ssum)[:, 0:1]
    hw = _ew(lambda a, s: a * s, h, sw)
    patt = _seg_sum(hw, batch2d, bpair["end_idx"], bpair["mask_end"])

    pooled = jnp.concatenate([pmean, pmax, psum, patt], axis=1)
    hdn = _mm(pooled, p["P1"].T, p["p1"],
              act=lambda y: jnp.maximum(y, 0.0))
    return _mm(hdn, p["P2"].T, p["p2"])


# speedup vs baseline: 1.3432x; 1.3432x over previous
import functools

import jax
import jax.numpy as jnp
import numpy as np
from jax import lax
from jax.experimental import pallas as pl
from jax.experimental.pallas import tpu as pltpu
from jax.experimental.pallas import tpu_sc as plsc

HID = 256
HEADS = 8
DH = HID // HEADS
G = 16

_NEG = -3.0e38

_M16 = np.zeros((HID, 16), np.float32)
for _h in range(HEADS):
    _M16[_h * DH:(_h + 1) * DH, _h] = 1.0
_M16S = _M16 / np.sqrt(DH)
_E16 = _M16.T.copy()


_SC_CHUNK = 128


@functools.lru_cache(maxsize=None)
def _sc_gather_fn(V, D, B):
    info = plsc.get_sparse_core_info()
    NC, NS = info.num_cores, info.num_subcores
    NW = NC * NS
    b_per_w = B // NW
    n_chunks = b_per_w // _SC_CHUNK
    mesh = plsc.VectorSubcoreMesh(core_axis_name="c", subcore_axis_name="s")

    @functools.partial(
        pl.kernel, mesh=mesh,
        out_type=jax.ShapeDtypeStruct((B, D), jnp.float32),
        scratch_types=[
            pltpu.VMEM((_SC_CHUNK,), jnp.int32),
            pltpu.VMEM((_SC_CHUNK, D), jnp.float32),
            pltpu.SemaphoreType.DMA,
        ],
    )
    def k(table_hbm, idx_hbm, out_hbm, idx_v, rows_v, sem):
        wid = lax.axis_index("s") * NC + lax.axis_index("c")
        base = wid * b_per_w

        def body(i, _):
            off = base + i * _SC_CHUNK
            pltpu.sync_copy(idx_hbm.at[pl.ds(off, _SC_CHUNK)], idx_v)
            pltpu.async_copy(table_hbm.at[idx_v], rows_v, sem).wait()
            pltpu.sync_copy(rows_v, out_hbm.at[pl.ds(off, _SC_CHUNK)])
            return 0

        lax.fori_loop(0, n_chunks, body, 0)

    return k


def _sc_gather(table, idx):
    V, D = table.shape
    B0 = idx.shape[0]
    B = ((B0 + 4095) // 4096) * 4096
    if B != B0:
        idx = jnp.concatenate([idx, jnp.zeros((B - B0,), jnp.int32)])
    out = _sc_gather_fn(V, D, B)(table, idx)
    return out[:B0]


_TM = 512


def _tm_for(M, pref):
    for t in (pref, 1000, 512):
        if t <= M and M % t == 0:
            return t
    return min(512, ((M + 7) // 8) * 8)


def _padrows(x, tm, val=0.0):
    m = x.shape[0]
    mp = ((m + tm - 1) // tm) * tm
    if mp == m:
        return x
    return jnp.pad(x, ((0, mp - m),) + ((0, 0),) * (x.ndim - 1),
                   constant_values=val)


def _ew(fn, *arrays):
    M = max(a.shape[0] for a in arrays)
    tm = _tm_for(M, 1000)
    padded = []
    specs = []
    shapes = []
    nblk = 1
    for a in arrays:
        if a.shape[0] == M:
            ap = _padrows(a, tm)
            nblk = ap.shape[0] // tm
            padded.append(ap)
            specs.append(pl.BlockSpec((tm, a.shape[1]), lambda i: (i, 0)))
            shapes.append((tm, a.shape[1]))
        else:
            padded.append(a)
            specs.append(pl.BlockSpec(a.shape, lambda i: (0, 0)))
            shapes.append(a.shape)
    oshape = jax.eval_shape(fn, *[
        jax.ShapeDtypeStruct(s, jnp.float32) for s in shapes])

    def body(*refs):
        ins = refs[:-1]
        out = refs[-1]
        out[...] = fn(*[r[...] for r in ins])

    res = pl.pallas_call(
        body,
        grid=(nblk,),
        in_specs=specs,
        out_specs=pl.BlockSpec((tm, oshape.shape[1]), lambda i: (i, 0)),
        out_shape=jax.ShapeDtypeStruct((nblk * tm, oshape.shape[1]),
                                       jnp.float32),
    )(*padded)
    return res[:M]


def _mm(x, wt, b=None, act=None):
    M, K = x.shape
    N = wt.shape[1]
    tm = _tm_for(M, 1000)
    xp = _padrows(x, tm)
    nblk = xp.shape[0] // tm
    if b is None:
        b = jnp.zeros((1, N), jnp.float32)
    else:
        b = b.reshape(1, N)

    def body(x_ref, w_ref, b_ref, o_ref):
        y = jnp.dot(x_ref[...], w_ref[...],
                    preferred_element_type=jnp.float32) + b_ref[...]
        if act is not None:
            y = act(y)
        o_ref[...] = y

    res = pl.pallas_call(
        body,
        grid=(nblk,),
        in_specs=[pl.BlockSpec((tm, K), lambda i: (i, 0)),
                  pl.BlockSpec((K, N), lambda i: (0, 0)),
                  pl.BlockSpec((1, N), lambda i: (0, 0))],
        out_specs=pl.BlockSpec((tm, N), lambda i: (i, 0)),
        out_shape=jax.ShapeDtypeStruct((nblk * tm, N), jnp.float32),
    )(xp, wt, b)
    return res[:M]


def _segscan(v, ids, kind, rev=False):
    M, D = v.shape
    tm = _tm_for(M, 4000 if D <= 16 else 1000)
    fill = 0.0 if kind == "add" else _NEG
    op = (lambda a, b: a + b) if kind == "add" else jnp.maximum
    vp = _padrows(v, tm, val=fill)
    idp = _padrows(ids, tm, val=np.int32(2147483647))
    nblk = vp.shape[0] // tm

    def imap(i):
        return ((nblk - 1 - i) if rev else i, 0)

    def body(v_ref, id_ref, o_ref, cm_ref, ci_ref):
        pid = pl.program_id(0)

        @pl.when(pid == 0)
        def _():
            cm_ref[...] = jnp.full((1, D), fill, jnp.float32)
            ci_ref[...] = jnp.full((1, 1), -1, jnp.int32)

        x = v_ref[...]
        sid = id_ref[...]
        k = 1
        while k < tm:
            if rev:
                xs = jnp.concatenate(
                    [x[k:], jnp.full((k, D), fill, jnp.float32)], axis=0)
                ss = jnp.concatenate(
                    [sid[k:], jnp.full((k, 1), -1, jnp.int32)], axis=0)
            else:
                xs = jnp.concatenate(
                    [jnp.full((k, D), fill, jnp.float32), x[:-k]], axis=0)
                ss = jnp.concatenate(
                    [jnp.full((k, 1), -1, jnp.int32), sid[:-k]], axis=0)
            x = jnp.where(sid == ss, op(x, xs), x)
            k *= 2
        x = jnp.where(sid == ci_ref[...], op(x, cm_ref[...]), x)
        o_ref[...] = x
        if rev:
            cm_ref[...] = x[0:1, :]
            ci_ref[...] = sid[0:1, :]
        else:
            cm_ref[...] = x[tm - 1:tm, :]
            ci_ref[...] = sid[tm - 1:tm, :]

    res = pl.pallas_call(
        body,
        grid=(nblk,),
        in_specs=[pl.BlockSpec((tm, D), imap),
                  pl.BlockSpec((tm, 1), imap)],
        out_specs=pl.BlockSpec((tm, D), imap),
        out_shape=jax.ShapeDtypeStruct((nblk * tm, D), jnp.float32),
        scratch_shapes=[pltpu.VMEM((1, D), jnp.float32),
                        pltpu.VMEM((1, 1), jnp.int32)],
    )(vp, idp)
    return res[:M]


def _colreduce(x, mode):
    M, D = x.shape
    tm = _tm_for(M, 1000)
    vp = _padrows(x, tm, val=(0.0 if mode == "sum2" else _NEG))
    nblk = vp.shape[0] // tm

    def body(v_ref, o_ref):
        pid = pl.program_id(0)

        @pl.when(pid == 0)
        def _():
            o_ref[...] = jnp.full(
                (8, D), 0.0 if mode == "sum2" else _NEG, jnp.float32)

        blk = v_ref[...]
        if mode == "sum2":
            o_ref[0:1, :] = o_ref[0:1, :] + jnp.sum(blk, 0, keepdims=True)
            o_ref[1:2, :] = o_ref[1:2, :] + jnp.sum(blk * blk, 0,
                                                    keepdims=True)
        else:
            o_ref[0:1, :] = jnp.maximum(o_ref[0:1, :],
                                        jnp.max(blk, 0, keepdims=True))

    res = pl.pallas_call(
        body,
        grid=(nblk,),
        in_specs=[pl.BlockSpec((tm, D), lambda i: (i, 0))],
        out_specs=pl.BlockSpec((8, D), lambda i: (0, 0)),
        out_shape=jax.ShapeDtypeStruct((8, D), jnp.float32),
    )(vp)
    return res


def _gate_kernel(out, xr, wbo, wbx):
    M = out.shape[0]
    tm = _tm_for(M, 1000)
    op = _padrows(out, tm)
    xp = _padrows(xr, tm)
    nblk = op.shape[0] // tm

    def body(o_ref, x_ref, wo_ref, wx_ref, y_ref):
        o = o_ref[...]
        x = x_ref[...]
        z = (jnp.dot(o, wo_ref[...], preferred_element_type=jnp.float32)
             + jnp.dot(x, wx_ref[...], preferred_element_type=jnp.float32))
        beta = jax.nn.sigmoid(z[:, 0:1])
        y_ref[...] = beta * x + (1.0 - beta) * o

    res = pl.pallas_call(
        body,
        grid=(nblk,),
        in_specs=[pl.BlockSpec((tm, HID), lambda i: (i, 0)),
                  pl.BlockSpec((tm, HID), lambda i: (i, 0)),
                  pl.BlockSpec((HID, 128), lambda i: (0, 0)),
                  pl.BlockSpec((HID, 128), lambda i: (0, 0))],
        out_specs=pl.BlockSpec((tm, HID), lambda i: (i, 0)),
        out_shape=jax.ShapeDtypeStruct((nblk * tm, HID), jnp.float32),
    )(op, xp, wbo, wbx)
    return res[:M]


def _seg_sum(vals, ids2d, end_idx, mask_end):
    sc = _segscan(vals, ids2d, "add")
    rows = _sc_gather(sc, end_idx)
    return _ew(lambda a, m: a * m, rows, mask_end)


def _softmax_norm(alpha16, dst2d):
    mf = _segscan(alpha16, dst2d, "max", rev=False)
    mb = _segscan(alpha16, dst2d, "max", rev=True)
    ex = _ew(lambda a, f, b: jnp.exp(a - jnp.maximum(f, b)),
             alpha16, mf, mb)
    sf = _segscan(ex, dst2d, "add", rev=False)
    sb = _segscan(ex, dst2d, "add", rev=True)
    return _ew(lambda e, f, b: e / (f + b - e + 1e-16), ex, sf, sb)


def kernel(x, edge_index, edge_attr, batch, params):
    p = params
    N = x.shape[0]
    E = edge_index.shape[1]
    mask16 = jnp.asarray(_M16)
    mask16s = jnp.asarray(_M16S)
    expand = jnp.asarray(_E16)

    dst0 = edge_index[1]
    perm = jnp.argsort(dst0).astype(jnp.int32)
    dst = dst0[perm]
    srcp = edge_index[0][perm]
    end = jnp.searchsorted(dst, jnp.arange(N, dtype=jnp.int32),
                           side="right").astype(jnp.int32)
    start = jnp.concatenate([jnp.zeros((1,), jnp.int32), end[:-1]])
    idx_pair = {
        "end_idx": jnp.maximum(end - 1, 0),
        "mask_end": (end > start).astype(jnp.float32).reshape(N, 1),
    }
    dst2d = dst.reshape(E, 1)

    bend = jnp.searchsorted(batch, jnp.arange(G, dtype=jnp.int32),
                            side="right").astype(jnp.int32)
    bstart = jnp.concatenate([jnp.zeros((1,), jnp.int32), bend[:-1]])
    bpair = {
        "end_idx": jnp.maximum(bend - 1, 0),
        "mask_end": (bend > bstart).astype(jnp.float32).reshape(G, 1),
    }
    batch2d = batch.astype(jnp.int32).reshape(N, 1)
    cnt = (bend - bstart).astype(jnp.float32).reshape(G, 1)

    xin = jnp.pad(x, ((0, 0), (0, 256 - x.shape[1])))
    encWt = jnp.pad(p["encW"].T, ((0, 256 - x.shape[1]), (0, 0)))
    h = _mm(xin, encWt, p["encb"])

    ea16 = jnp.pad(edge_attr, ((0, 0), (0, 16 - edge_attr.shape[1])))
    eencWt = jnp.pad(p["eencW"].T, ((0, 16 - edge_attr.shape[1]), (0, 0)))
    ea0 = _mm(ea16, eencWt, p["eencb"])
    ea = _sc_gather(ea0, perm)

    def tconv(xc, pre):
        q = _mm(xc, p[pre + "Wq"].T, p[pre + "bq"])
        k = _mm(xc, p[pre + "Wk"].T, p[pre + "bk"])
        v = _mm(xc, p[pre + "Wv"].T, p[pre + "bv"])
        e = _mm(ea, p[pre + "We"].T)
        kv = jnp.concatenate([k, v], axis=1)
        kvs = _sc_gather(kv, srcp)
        ks, vs = kvs[:, :HID], kvs[:, HID:]
        qd = _sc_gather(q, dst)
        alpha16 = _ew(lambda a, b, c, mk: jnp.dot(
            (a * (b + c)), mk, preferred_element_type=jnp.float32),
            qd, ks, e, mask16s)
        alphan = _softmax_norm(alpha16, dst2d)
        wv = _ew(lambda vv, ee, an, ex16: (vv + ee) * jnp.dot(
            an, ex16, preferred_element_type=jnp.float32),
            vs, e, alphan, expand)
        out = _seg_sum(wv, dst2d, idx_pair["end_idx"],
                       idx_pair["mask_end"])
        xr = _mm(xc, p[pre + "Ws"].T, p[pre + "bs"])
        wb = p[pre + "Wb"]
        wbo = jnp.pad((wb[:, :HID] + wb[:, 2 * HID:]).T, ((0, 0), (0, 127)))
        wbx = jnp.pad((wb[:, HID:2 * HID] - wb[:, 2 * HID:]).T,
                      ((0, 0), (0, 127)))
        return _gate_kernel(out, xr, wbo, wbx)

    def graphnorm(xc, w, b, ms):
        ss = _colreduce(xc, "sum2")
        m = ss[0] / xc.shape[0]
        ex2 = ss[1] / xc.shape[0]
        var = ex2 - 2.0 * ms * m * m + ms * ms * m * m
        scale = w / jnp.sqrt(var + 1e-5)
        shift = b - ms * m * scale
        return _ew(lambda u, a, c: u * a + c,
                   xc, scale.reshape(1, HID), shift.reshape(1, HID))

    def gt_layer(xc, pre):
        xa = tconv(xc, pre)
        x1 = _ew(lambda a, b: a + b, xc, xa)
        xg = graphnorm(x1, p[pre + "nw"], p[pre + "nb"], p[pre + "nms"])
        f = _mm(xg, p[pre + "F1"].T, p[pre + "f1"], act=jax.nn.gelu)
        f2 = _mm(f, p[pre + "F2"].T, p[pre + "f2"])
        return _ew(lambda a, b: a + b, xg, f2)

    def gat(xc, pre):
        hx = _mm(xc, p[pre + "W"].T)
        e = _mm(ea, p[pre + "We"].T)
        asf = p[pre + "as"].reshape(1, HID)
        adf = p[pre + "ad"].reshape(1, HID)
        aef = p[pre + "ae"].reshape(1, HID)
        hdot = lambda hh, aa, mk: jnp.dot(
            hh * aa, mk, preferred_element_type=jnp.float32)
        asn = _ew(hdot, hx, asf, mask16)
        adn = _ew(hdot, hx, adf, mask16)
        aen = _ew(hdot, e, aef, mask16)
        src_tab = jnp.concatenate(
            [hx, jnp.pad(asn, ((0, 0), (0, 112)))], axis=1)
        g = _sc_gather(src_tab, srcp)
        hxs = g[:, :HID]
        g_s = g[:, HID:HID + 16]
        g_d = _sc_gather(jnp.pad(adn, ((0, 0), (0, 112))), dst)[:, :16]
        alpha16 = _ew(lambda a, b, c: jax.nn.leaky_relu(a + b + c, 0.2),
                      g_s, g_d, aen)
        alphan = _softmax_norm(alpha16, dst2d)
        wv = _ew(lambda vv, an, ex16: vv * jnp.dot(
            an, ex16, preferred_element_type=jnp.float32),
            hxs, alphan, expand)
        out = _seg_sum(wv, dst2d, idx_pair["end_idx"],
                       idx_pair["mask_end"])
        return _ew(lambda a, b: a + b, out, p[pre + "bias"].reshape(1, HID))

    def gin(xc, pre):
        xs = _sc_gather(xc, srcp)
        agg = _seg_sum(xs, dst2d, idx_pair["end_idx"],
                       idx_pair["mask_end"])
        eps11 = (1.0 + p[pre + "eps"]).reshape(1, 1)
        z = _ew(lambda a, b, c: a * c + b, xc, agg, eps11)
        z = _mm(z, p[pre + "W1"].T, p[pre + "b1"])
        ss = _colreduce(z, "sum2")
        m = ss[0] / z.shape[0]
        var = ss[1] / z.shape[0] - m * m
        scale = p[pre + "bnw"] / jnp.sqrt(var + 1e-5)
        shift = p[pre + "bnb"] - m * scale
        z = _ew(lambda u, a, c: jnp.maximum(u * a + c, 0.0),
                z, scale.reshape(1, 2 * HID), shift.reshape(1, 2 * HID))
        return _mm(z, p[pre + "W2"].T, p[pre + "b2"])

    for i in range(6):
        pre = "L%d_" % i
        if i % 3 == 0:
            xn = gt_layer(h, pre)
        elif i % 3 == 1:
            xn = gat(h, pre)
        else:
            xn = gin(h, pre)
        h = _ew(lambda a, b: a + b, h, xn)
        h = graphnorm(h, p["N%d_w" % i], p["N%d_b" % i], p["N%d_ms" % i])

    psum = _seg_sum(h, batch2d, bpair["end_idx"], bpair["mask_end"])
    pmean = _ew(lambda a, c: a / jnp.maximum(c, 1.0), psum, cnt)
    mscan = _segscan(h, batch2d, "max", rev=False)
    pmax = _sc_gather(mscan, bpair["end_idx"])
    pmax = _ew(lambda m, msk: jnp.where(msk > 0, m, 0.0),
               pmax, bpair["mask_end"])

    t = _mm(h, p["A1"].T, p["a1"], act=jnp.tanh)
    a2t = jnp.pad(p["A2"].T, ((0, 0), (0, 127)))
    a2b = jnp.pad(p["a2"].reshape(1, 1), ((0, 0), (0, 127)))
    spre = _mm(t, a2t, a2b[0])
    cm = _colreduce(spre, "max")[0:1]
    exs = _ew(lambda a, m: jnp.exp(a - m), spre, cm)
    ssum = _colreduce(exs, "sum2")[0:1]
    sw = _ew(lambda a, s: a / s, exs, ssum)[:, 0:1]
    hw = _ew(lambda a, s: a * s, h, sw)
    patt = _seg_sum(hw, batch2d, bpair["end_idx"], bpair["mask_end"])

    pooled = jnp.concatenate([pmean, pmax, psum, patt], axis=1)
    hdn = _mm(pooled, p["P1"].T, p["p1"],
              act=lambda y: jnp.maximum(y, 0.0))
    return _mm(hdn, p["P2"].T, p["p2"])
